# baseline (speedup 1.0000x reference)
import functools

import jax
import jax.numpy as jnp
from jax.experimental import pallas as pl
from jax.experimental.pallas import tpu as pltpu

_TOPK = 64
_KT = 4096
_CHUNKS = _KT // 128
_NEG_INF = float("-inf")
_BIG_I = 2**30


def _rank_gt(av, ai, bv, bi):
    return (av > bv) | ((av == bv) & (ai < bi))


def _insert_one(rv, ri, mv, gi, lane64):
    beat = _rank_gt(mv, gi, rv[:, _TOPK - 1], ri[:, _TOPK - 1])
    ge = _rank_gt(mv[:, None], gi[:, None], rv, ri) & beat[:, None]
    ge_prev = (jnp.roll(ge.astype(jnp.int32), 1, axis=1) != 0) & (lane64 > 0)
    rv = jnp.where(ge, jnp.where(ge_prev, jnp.roll(rv, 1, axis=1), mv[:, None]), rv)
    ri = jnp.where(ge, jnp.where(ge_prev, jnp.roll(ri, 1, axis=1), gi[:, None]), ri)
    return rv, ri


def _candidate_insert_loop(cv, cidx, rv, ri, lane64):
    def row_best(cv):
        mv = jnp.max(cv, axis=1)
        hit = cv == mv[:, None]
        gi = jnp.min(jnp.where(hit, cidx, _BIG_I), axis=1)
        return mv, gi

    def cond(state):
        _, rv, ri, mv, gi = state
        return jnp.any(_rank_gt(mv, gi, rv[:, _TOPK - 1], ri[:, _TOPK - 1]))

    def body(state):
        cv, rv, ri, mv, gi = state
        rv, ri = _insert_one(rv, ri, mv, gi, lane64)
        cv = jnp.where(cidx == gi[:, None], _NEG_INF, cv)
        mv, gi = row_best(cv)
        return cv, rv, ri, mv, gi

    mv0, gi0 = row_best(cv)
    _, rv, ri, _, _ = jax.lax.while_loop(cond, body, (cv, rv, ri, mv0, gi0))
    return rv, ri


def _knn_kernel(nsteps, nkeys, q_ref, k_ref, qsq_ref, ksq_ref,
                vals_ref, idx_ref, rv_ref, ri_ref):
    step = pl.program_id(0)
    nq = q_ref.shape[0]

    @pl.when(step == 0)
    def _init():
        rv_ref[...] = jnp.full((nq, _TOPK), _NEG_INF, jnp.float32)
        ri_ref[...] = jnp.zeros((nq, _TOPK), jnp.int32)

    q = q_ref[...]
    k = k_ref[...]
    dots = jax.lax.dot_general(
        q.astype(jnp.bfloat16), k.astype(jnp.bfloat16),
        (((1,), (1,)), ((), ())), preferred_element_type=jnp.float32)
    q_sq = qsq_ref[...]
    k_sq = ksq_ref[...]
    negd2 = -(q_sq + k_sq - 2.0 * dots)

    base = step * _KT
    lane_t = jax.lax.broadcasted_iota(jnp.int32, (1, _KT), 1)
    valid = (base + lane_t) < nkeys
    v = jnp.where(valid, negd2, _NEG_INF)

    lane128 = jax.lax.broadcasted_iota(jnp.int32, (nq, 128), 1)
    lane64 = jax.lax.broadcasted_iota(jnp.int32, (nq, _TOPK), 1)

    sl = [v[:, c * 128:(c + 1) * 128] for c in range(_CHUNKS)]

    rv = rv_ref[...]
    ri = ri_ref[...]

    m1 = functools.reduce(jnp.maximum, sl)
    c1 = functools.reduce(
        jnp.minimum,
        [jnp.where(s == m1, c, _CHUNKS) for c, s in enumerate(sl)])
    i1 = base + c1 * 128 + lane128
    rv, ri = _candidate_insert_loop(m1, i1, rv, ri, lane64)

    slx = [jnp.where(c1 == c, _NEG_INF, s) for c, s in enumerate(sl)]
    m2 = functools.reduce(jnp.maximum, slx)
    c2 = functools.reduce(
        jnp.minimum,
        [jnp.where(s == m2, c, _CHUNKS) for c, s in enumerate(slx)])
    i2 = base + c2 * 128 + lane128
    rv, ri = _candidate_insert_loop(m2, i2, rv, ri, lane64)

    rv_ref[...] = rv
    ri_ref[...] = ri

    m3 = functools.reduce(
        jnp.maximum,
        [jnp.where(c2 == c, _NEG_INF, s) for c, s in enumerate(slx)])
    need_deep = jnp.any(m3 >= rv[:, _TOPK - 1][:, None])

    @pl.when(need_deep)
    def _deep():
        rvd = rv_ref[...]
        rid = ri_ref[...]
        gidx = [base + c * 128 + lane128 for c in range(_CHUNKS)]

        def row_best(slr):
            m = functools.reduce(jnp.maximum, slr)
            mv = jnp.max(m, axis=1)
            gi = functools.reduce(
                jnp.minimum,
                [jnp.min(jnp.where(s == mv[:, None], g, _BIG_I), axis=1)
                 for s, g in zip(slr, gidx)])
            return mv, gi

        def cond(state):
            rvd, rid, mv, gi = state[_CHUNKS:]
            return jnp.any(_rank_gt(mv, gi, rvd[:, _TOPK - 1], rid[:, _TOPK - 1]))

        def body(state):
            slr = list(state[:_CHUNKS])
            rvd, rid, mv, gi = state[_CHUNKS:]
            rvd, rid = _insert_one(rvd, rid, mv, gi, lane64)
            slr = [jnp.where(g == gi[:, None], _NEG_INF, s)
                   for s, g in zip(slr, gidx)]
            mv, gi = row_best(slr)
            return tuple(slr) + (rvd, rid, mv, gi)

        sly = [jnp.where(c2 == c, _NEG_INF, s) for c, s in enumerate(slx)]
        mv0, gi0 = row_best(sly)
        out = jax.lax.while_loop(
            cond, body, tuple(sly) + (rvd, rid, mv0, gi0))
        rv_ref[...] = out[_CHUNKS]
        ri_ref[...] = out[_CHUNKS + 1]

    @pl.when(step == nsteps - 1)
    def _emit():
        vals_ref[...] = rv_ref[...]
        idx_ref[...] = ri_ref[...]


def kernel(queries, keys):
    nq, d = queries.shape
    nkeys = keys.shape[0]
    nsteps = -(-nkeys // _KT)
    pad = nsteps * _KT - nkeys
    keys_p = jnp.pad(keys, ((0, pad), (0, 0))) if pad else keys

    q_sq = jnp.sum(queries * queries, axis=1, keepdims=True)
    k_sq = jnp.sum(keys_p * keys_p, axis=1)[None, :]

    vals, idx = pl.pallas_call(
        functools.partial(_knn_kernel, nsteps, nkeys),
        grid=(nsteps,),
        in_specs=[
            pl.BlockSpec((nq, d), lambda s: (0, 0)),
            pl.BlockSpec((_KT, d), lambda s: (s, 0)),
            pl.BlockSpec((nq, 1), lambda s: (0, 0)),
            pl.BlockSpec((1, _KT), lambda s: (0, s)),
        ],
        out_specs=[
            pl.BlockSpec((nq, _TOPK), lambda s: (0, 0)),
            pl.BlockSpec((nq, _TOPK), lambda s: (0, 0)),
        ],
        out_shape=[
            jax.ShapeDtypeStruct((nq, _TOPK), jnp.float32),
            jax.ShapeDtypeStruct((nq, _TOPK), jnp.int32),
        ],
        scratch_shapes=[
            pltpu.VMEM((nq, _TOPK), jnp.float32),
            pltpu.VMEM((nq, _TOPK), jnp.int32),
        ],
        compiler_params=pltpu.CompilerParams(
            dimension_semantics=("arbitrary",)),
    )(queries, keys_p, q_sq, k_sq)
    return vals, idx
